# baseline (speedup 1.0000x reference)
import jax
import jax.numpy as jnp
from jax import lax
from jax.experimental import pallas as pl
from jax.experimental.pallas import tpu as pltpu

_CHUNK = 2048


def _outer(x_ref, w0_ref, b0_ref, w1_ref, b1_ref, w2_ref, b2_ref,
           w3t_ref, b3_ref, out_ref):
    def body(x_blk, out_blk):
        h = x_blk[...]
        for w_ref, b_ref in ((w0_ref, b0_ref), (w1_ref, b1_ref),
                             (w2_ref, b2_ref)):
            h = jnp.dot(h, w_ref[...],
                        preferred_element_type=jnp.float32) + b_ref[...]
            h = 0.5 * h * (1.0 + jnp.tanh(0.5 * h))
        o = lax.dot_general(h, w3t_ref[...], (((1,), (1,)), ((), ())),
                            preferred_element_type=jnp.float32) + b3_ref[...]
        ot = o.T
        m = jnp.max(ot, axis=0, keepdims=True)
        s = ot - m
        lse = jnp.log(jnp.sum(jnp.exp(s), axis=0, keepdims=True))
        out_blk[...] = s - lse

    n, d = x_ref.shape
    n_out = out_ref.shape[0]
    n_main = (n // _CHUNK) * _CHUNK
    n_tail = n - n_main
    pipe = pltpu.emit_pipeline(
        body,
        grid=(n_main // _CHUNK,),
        in_specs=[pl.BlockSpec((_CHUNK, d), lambda i: (i, 0))],
        out_specs=[pl.BlockSpec((n_out, _CHUNK), lambda i: (0, i))],
    )
    pipe(x_ref, out_ref)
    if n_tail:
        tail = pltpu.emit_pipeline(
            body,
            grid=(1,),
            in_specs=[pl.BlockSpec((n_tail, d), lambda i: (0, 0))],
            out_specs=[pl.BlockSpec((n_out, n_tail), lambda i: (0, 0))],
        )
        tail(x_ref.at[pl.ds(n_main, n_tail), :],
             out_ref.at[:, pl.ds(n_main, n_tail)])


def kernel(x, edge_index, W0, b0, W1, b1, W2, b2, W3, b3):
    del edge_index
    n, d = x.shape
    n_out = W3.shape[1]

    def full(arr):
        return pl.BlockSpec(arr.shape, lambda: (0,) * arr.ndim)

    W3t = W3.T
    out_t = pl.pallas_call(
        _outer,
        in_specs=[
            pl.BlockSpec(memory_space=pl.ANY),
            full(W0), full(b0), full(W1), full(b1),
            full(W2), full(b2), full(W3t), full(b3),
        ],
        out_specs=pl.BlockSpec(memory_space=pl.ANY),
        out_shape=jax.ShapeDtypeStruct((n_out, n), x.dtype),
    )(x, W0, b0, W1, b1, W2, b2, W3t, b3)
    return out_t.T


# speedup vs baseline: 1.4588x; 1.4588x over previous
import jax
import jax.numpy as jnp
from jax import lax
from jax.experimental import pallas as pl

_HALF = 2560
_BLOCK = 2 * _HALF


def _mlp_half(x_ref, w_refs, b_refs, w3t_ref, b3_ref):
    h = x_ref[...]
    for w_ref, b_ref in zip(w_refs, b_refs):
        h = jnp.dot(h, w_ref[...], preferred_element_type=jnp.float32) + b_ref[...]
        h = 0.5 * h * (1.0 + jnp.tanh(0.5 * h))
    o = lax.dot_general(h, w3t_ref[...], (((1,), (1,)), ((), ())),
                        preferred_element_type=jnp.float32) + b3_ref[...]
    ot = o.T
    m = jnp.max(ot, axis=0, keepdims=True)
    s = ot - m
    lse = jnp.log(jnp.sum(jnp.exp(s), axis=0, keepdims=True))
    return s - lse


def _fused_mlp_kernel(x0_ref, x1_ref, w0_ref, b0_ref, w1_ref, b1_ref,
                      w2_ref, b2_ref, w3t_ref, b3_ref, out_ref):
    w_refs = (w0_ref, w1_ref, w2_ref)
    b_refs = (b0_ref, b1_ref, b2_ref)
    out_ref[:, :_HALF] = _mlp_half(x0_ref, w_refs, b_refs, w3t_ref, b3_ref)
    out_ref[:, _HALF:] = _mlp_half(x1_ref, w_refs, b_refs, w3t_ref, b3_ref)


def kernel(x, edge_index, W0, b0, W1, b1, W2, b2, W3, b3):
    del edge_index
    n, d = x.shape
    n_out = W3.shape[1]
    grid = ((n + _BLOCK - 1) // _BLOCK,)

    def full(arr):
        return pl.BlockSpec(arr.shape, lambda i: (0,) * arr.ndim)

    W3t = W3.T
    out_t = pl.pallas_call(
        _fused_mlp_kernel,
        grid=grid,
        in_specs=[
            pl.BlockSpec((_HALF, d), lambda i: (2 * i, 0)),
            pl.BlockSpec((_HALF, d), lambda i: (2 * i + 1, 0)),
            full(W0), full(b0), full(W1), full(b1),
            full(W2), full(b2), full(W3t), full(b3),
        ],
        out_specs=pl.BlockSpec((n_out, _BLOCK), lambda i: (0, i)),
        out_shape=jax.ShapeDtypeStruct((n_out, n), x.dtype),
    )(x, x, W0, b0, W1, b1, W2, b2, W3t, b3)
    return out_t.T


# speedup vs baseline: 1.4827x; 1.0164x over previous
import jax
import jax.numpy as jnp
from jax import lax
from jax.experimental import pallas as pl

_BLOCK = 5120


def _fused_mlp_kernel(x_ref, w0_ref, b0_ref, w1_ref, b1_ref, w2_ref, b2_ref,
                      w3t_ref, b3_ref, out_ref):
    h = x_ref[...]
    for w_ref, b_ref in ((w0_ref, b0_ref), (w1_ref, b1_ref), (w2_ref, b2_ref)):
        h = jnp.dot(h, w_ref[...], preferred_element_type=jnp.float32) + b_ref[...]
        h = 0.5 * h * (1.0 + jnp.tanh(0.5 * h))
    o = lax.dot_general(h, w3t_ref[...], (((1,), (1,)), ((), ())),
                        preferred_element_type=jnp.float32) + b3_ref[...]
    ot = o.T
    m = jnp.max(ot, axis=0, keepdims=True)
    s = ot - m
    lse = jnp.log(jnp.sum(jnp.exp(s), axis=0, keepdims=True))
    out_ref[...] = s - lse


def kernel(x, edge_index, W0, b0, W1, b1, W2, b2, W3, b3):
    del edge_index
    n, d = x.shape
    n_out = W3.shape[1]
    grid = ((n + _BLOCK - 1) // _BLOCK,)

    def full(arr):
        return pl.BlockSpec(arr.shape, lambda i: (0,) * arr.ndim)

    W3t = W3.T
    out_t = pl.pallas_call(
        _fused_mlp_kernel,
        grid=grid,
        in_specs=[
            pl.BlockSpec((_BLOCK, d), lambda i: (i, 0)),
            full(W0), full(b0), full(W1), full(b1),
            full(W2), full(b2), full(W3t), full(b3),
        ],
        out_specs=pl.BlockSpec((n_out, _BLOCK), lambda i: (0, i)),
        out_shape=jax.ShapeDtypeStruct((n_out, n), x.dtype),
    )(x, W0, b0, W1, b1, W2, b2, W3t, b3)
    return out_t.T
